# baseline (speedup 1.0000x reference)
import functools

import jax
import jax.numpy as jnp
from jax import lax
from jax.experimental import pallas as pl
from jax.experimental.pallas import tpu as pltpu
from jax.experimental.pallas import tpu_sc as plsc

N = 10000
NP = 10240
E = 320000
F = 128
NC = 2
NS = 16
NW = NC * NS
CH = 40
NCH = 250
EPW = NCH * CH
SIB = 25
NBLK = NCH // SIB
ND = 5
RPT = NP // NS
NPA = 10112
RPA = NPA // NS

_mesh = plsc.VectorSubcoreMesh(core_axis_name="c", subcore_axis_name="s")


def _zero_vmem(ref, nvec):
    z = jnp.zeros((16,), jnp.float32)

    def body(i, _):
        ref[pl.ds(i * 16, 16)] = z
        return 0

    lax.fori_loop(0, nvec, body, 0)


@functools.partial(
    pl.kernel,
    out_type=jax.ShapeDtypeStruct((NC, NP), jnp.float32),
    mesh=_mesh,
    scratch_types=[
        pltpu.VMEM((EPW,), jnp.int32),
        pltpu.VMEM((NP,), jnp.float32),
        pltpu.VMEM((RPT,), jnp.float32),
        pltpu.VMEM((NS, RPT), jnp.float32),
        pltpu.VMEM_SHARED((NS, NP), jnp.float32),
        pltpu.SemaphoreType.DMA,
    ],
    compiler_params=pltpu.CompilerParams(needs_layout_passes=False),
)
def _deg_kernel(dst_hbm, deg_out, idx_v, deg_v, sum_v, tmp_v, stage_s, sem):
    cid = lax.axis_index("c")
    sid = lax.axis_index("s")
    wid = cid * NS + sid

    pltpu.async_copy(dst_hbm.at[wid], idx_v, sem)
    _zero_vmem(deg_v, NP // 16)
    pltpu.make_async_copy(dst_hbm.at[wid], idx_v, sem).wait()

    ones = jnp.full((16,), 1.0, jnp.float32)

    def body(g, _):
        for u in range(5):
            iv = idx_v[pl.ds((g * 5 + u) * 16, 16)]
            plsc.addupdate_scatter(deg_v, [iv], ones)
        return 0

    lax.fori_loop(0, EPW // 80, body, 0)

    pltpu.sync_copy(deg_v, stage_s.at[sid])
    plsc.subcore_barrier()

    base = sid * RPT
    pltpu.sync_copy(stage_s.at[:, pl.ds(base, RPT)], tmp_v)

    def add(j, _):
        sl = pl.ds(j * 16, 16)
        acc = tmp_v[0, sl]
        for t in range(1, NS):
            acc = acc + tmp_v[t, sl]
        sum_v[sl] = acc
        return 0

    lax.fori_loop(0, RPT // 16, add, 0)
    pltpu.sync_copy(sum_v, deg_out.at[cid, pl.ds(base, RPT)])


@functools.partial(
    pl.kernel,
    out_type=jax.ShapeDtypeStruct((NC, NPA, F), jnp.float32),
    mesh=_mesh,
    scratch_types=[
        pltpu.VMEM((2, SIB, CH), jnp.int32),
        pltpu.VMEM((2, SIB, CH), jnp.int32),
        pltpu.VMEM((CH, F), jnp.float32),
        pltpu.VMEM((CH, F), jnp.float32),
        pltpu.VMEM((CH, F), jnp.float32),
        pltpu.VMEM((CH, F), jnp.float32),
        pltpu.VMEM((CH, F), jnp.float32),
        pltpu.VMEM_SHARED((NPA, F), jnp.float32),
        pltpu.SemaphoreType.DMA,
        pltpu.SemaphoreType.DMA,
        pltpu.SemaphoreType.DMA,
        pltpu.SemaphoreType.DMA,
        pltpu.SemaphoreType.DMA,
        pltpu.SemaphoreType.DMA,
    ],
)
def _agg_kernel(src_hbm, dst_hbm, feat_hbm, acc_out,
                si_v, di_v, rows_0, rows_1, rows_2, rows_3, rows_4, acc_s,
                sem_0, sem_1, sem_2, sem_3, sem_4, sem_i):
    cid = lax.axis_index("c")
    sid = lax.axis_index("s")
    wid = cid * NS + sid

    z = jnp.zeros((16,), jnp.float32)

    def zrow(r, _):
        for j in range(F // 16):
            rows_4[r, pl.ds(j * 16, 16)] = z
        return 0

    lax.fori_loop(0, CH, zrow, 0)

    def init_descs():
        descs = []
        for r in range(RPA // CH):
            descs.append((rows_4, acc_s.at[pl.ds(sid * RPA + r * CH, CH)]))
        rem = RPA % CH
        if rem:
            descs.append((rows_4.at[pl.ds(0, rem)],
                          acc_s.at[pl.ds(sid * RPA + (RPA // CH) * CH, rem)]))
        return descs

    for s, d in init_descs():
        pltpu.async_copy(s, d, sem_4)

    def stage_start(b):
        pltpu.async_copy(src_hbm.at[wid, b], si_v.at[b % 2], sem_i)
        pltpu.async_copy(dst_hbm.at[wid, b], di_v.at[b % 2], sem_i)

    def stage_wait(b):
        pltpu.make_async_copy(src_hbm.at[wid, b], si_v.at[b % 2], sem_i).wait()
        pltpu.make_async_copy(dst_hbm.at[wid, b], di_v.at[b % 2], sem_i).wait()

    def g_start(b, r, buf, sem):
        pltpu.async_copy(feat_hbm.at[si_v.at[b % 2, r]], buf, sem)

    def g_wait(b, r, buf, sem):
        pltpu.make_async_copy(feat_hbm.at[si_v.at[b % 2, r]], buf, sem).wait()

    def s_add(b, r, buf):
        pltpu.sync_copy(buf, acc_s.at[di_v.at[b % 2, r]], add=True)

    bufs = (rows_0, rows_1, rows_2, rows_3, rows_4)
    sems = (sem_0, sem_1, sem_2, sem_3, sem_4)

    stage_start(0)
    stage_wait(0)
    for j in range(ND - 1):
        g_start(0, j, bufs[j], sems[j])
    for s, d in init_descs():
        pltpu.make_async_copy(s, d, sem_4).wait()
    plsc.subcore_barrier()
    g_start(0, ND - 1, bufs[ND - 1], sems[ND - 1])

    for b in range(NBLK):
        if b + 1 < NBLK:
            stage_start(b + 1)
        rot = (SIB * b) % ND
        b4 = tuple(bufs[(rot + j) % ND] for j in range(ND))
        s4 = tuple(sems[(rot + j) % ND] for j in range(ND))

        def body(m, _, b=b, b4=b4, s4=s4):
            l0 = ND * m
            for j in range(ND):
                g_wait(b, l0 + j, b4[j], s4[j])
                s_add(b, l0 + j, b4[j])
                g_start(b, l0 + j + ND, b4[j], s4[j])
            return 0

        nfull = (SIB - ND - (ND - 1) - 1) // ND + 1
        lax.fori_loop(0, nfull, body, 0)

        for l in range(ND * nfull, SIB):
            bf = b4[l % ND]
            sm = s4[l % ND]
            g_wait(b, l, bf, sm)
            s_add(b, l, bf)
            nxt = l + ND
            if nxt < SIB:
                g_start(b, nxt, bf, sm)
            elif b + 1 < NBLK:
                if nxt == SIB:
                    stage_wait(b + 1)
                g_start(b + 1, nxt - SIB, bf, sm)

    plsc.subcore_barrier()
    pltpu.sync_copy(acc_s.at[pl.ds(sid * RPA, RPA)],
                    acc_out.at[cid, pl.ds(sid * RPA, RPA)])


_GRID = 10
_BR = N // _GRID


def _p2_body(degt_ref, x_ref, dis_ref, xs_ref):
    deg = degt_ref[:, 0:1] + degt_ref[:, 1:2] + 1.0
    dis = lax.rsqrt(deg)
    dis_ref[...] = dis
    xs_ref[...] = dis * x_ref[...]


def _p4_body(acc_ref, xs_ref, dis_ref, w1_ref, b1_ref, w2_ref, gs_ref):
    z1 = dis_ref[...] * (acc_ref[0] + acc_ref[1] + xs_ref[...])
    h = jnp.dot(z1, w1_ref[...], preferred_element_type=jnp.float32)
    h = jnp.maximum(h + b1_ref[...], 0.0)
    g = jnp.dot(h, w2_ref[...], preferred_element_type=jnp.float32)
    gs_ref[...] = dis_ref[...] * g


def _p6_body(acc_ref, gs_ref, dis_ref, b2_ref, out_ref):
    out_ref[...] = (dis_ref[...] * (acc_ref[0] + acc_ref[1] + gs_ref[...])
                    + b2_ref[...])


def _rows(i):
    return (i, 0)


def _full(i):
    return (0, 0)


_p2_call = pl.pallas_call(
    _p2_body,
    grid=(_GRID,),
    in_specs=[
        pl.BlockSpec((_BR, 2), _rows),
        pl.BlockSpec((_BR, F), _rows),
    ],
    out_specs=[
        pl.BlockSpec((_BR, 1), _rows),
        pl.BlockSpec((_BR, F), _rows),
    ],
    out_shape=[
        jax.ShapeDtypeStruct((N, 1), jnp.float32),
        jax.ShapeDtypeStruct((N, F), jnp.float32),
    ],
)

_p4_call = pl.pallas_call(
    _p4_body,
    grid=(_GRID,),
    in_specs=[
        pl.BlockSpec((NC, _BR, F), lambda i: (0, i, 0)),
        pl.BlockSpec((_BR, F), _rows),
        pl.BlockSpec((_BR, 1), _rows),
        pl.BlockSpec((F, 2 * F), _full),
        pl.BlockSpec((1, 2 * F), _full),
        pl.BlockSpec((2 * F, F), _full),
    ],
    out_specs=pl.BlockSpec((_BR, F), _rows),
    out_shape=jax.ShapeDtypeStruct((N, F), jnp.float32),
)

_p6_call = pl.pallas_call(
    _p6_body,
    grid=(_GRID,),
    in_specs=[
        pl.BlockSpec((NC, _BR, F), lambda i: (0, i, 0)),
        pl.BlockSpec((_BR, F), _rows),
        pl.BlockSpec((_BR, 1), _rows),
        pl.BlockSpec((1, F), _full),
    ],
    out_specs=pl.BlockSpec((_BR, F), _rows),
    out_shape=jax.ShapeDtypeStruct((N, F), jnp.float32),
)


def kernel(x, edge_index, W1, b1, W2, b2):
    ei = edge_index.astype(jnp.int32)
    src3 = ei[0].reshape(NW, NBLK, SIB, CH)
    dst3 = ei[1].reshape(NW, NBLK, SIB, CH)
    dst2 = ei[1].reshape(NW, EPW)

    degp = _deg_kernel(dst2)
    degt = degp.T[:N]
    dis, xs = _p2_call(degt, x)
    acc1 = _agg_kernel(src3, dst3, xs)
    gs = _p4_call(acc1, xs, dis, W1, b1.reshape(1, -1), W2)
    acc2 = _agg_kernel(src3, dst3, gs)
    return _p6_call(acc2, gs, dis, b2.reshape(1, -1))
